# baseline (speedup 1.0000x reference)
import functools

import jax
import jax.numpy as jnp
from jax import lax
from jax.experimental import pallas as pl
from jax.experimental.pallas import tpu as pltpu
from jax.experimental.pallas import tpu_sc as plsc

_N = 10000
_E = 320000
_D = 128
_H = _D // 2
_OUT = 128

_NC = 2
_NS = 16

_R = 1024
_NPAD = ((_N + _R - 1) // _R) * _R
_G = 40
_NB = 5
_GRP = _NB * _G
_S = 4
_EPT = _E // _NS
_NKG = _EPT // _GRP
_NSG = _NKG // _S
_RPT = _NPAD // _NS


def _sc_body(xh_hbm, idx_hbm, out_hbm, idxb, rows, acc, gsem):
    cid = lax.axis_index("c")
    sid = lax.axis_index("s")
    xh = xh_hbm.at[cid]

    z16 = jnp.zeros((16,), jnp.float32)

    def _zrow(r, carry):
        def _zcol(c, inner):
            rows[r, pl.ds(c * 16, 16)] = z16
            return inner
        return lax.fori_loop(0, _H // 16, _zcol, carry)

    lax.fori_loop(0, _G, _zrow, 0)

    tbase = sid * _RPT

    def _zacc(k, carry):
        pltpu.sync_copy(rows.at[pl.ds(0, _G)], acc.at[pl.ds(tbase + k * _G, _G)])
        return carry

    lax.fori_loop(0, _RPT // _G, _zacc, 0)
    plsc.subcore_barrier()

    def _slot(p, b):
        return rows.at[pl.ds((p * _NB + b) * _G, _G)]

    def _ld_sg(s, q):
        pltpu.sync_copy(idx_hbm.at[sid, s], idxb.at[q])

    def _gathers(q, j, p):
        for b in range(_NB):
            pltpu.async_copy(xh.at[idxb.at[q, j, b, 0]], _slot(p, b),
                             gsem.at[p * _NB + b])

    def _wait_gathers(q, j, p):
        for b in range(_NB):
            pltpu.make_async_copy(xh.at[idxb.at[q, j, b, 0]], _slot(p, b),
                                  gsem.at[p * _NB + b]).wait()

    def _scatters(q, j, p):
        for b in range(_NB):
            pltpu.sync_copy(_slot(p, b), acc.at[idxb.at[q, j, b, 1]], add=True)

    def _steady(cur, nxt, load=None):
        (q, j, p), (q2, j2) = cur, nxt
        if load is not None:
            _ld_sg(load, q2)
        _gathers(q2, j2, 1 - p)
        _wait_gathers(q, j, p)
        _scatters(q, j, p)

    _ld_sg(0, 0)
    _gathers(0, 0, 0)

    def _sg_pair(k, carry):
        s = 2 * k
        for q in (0, 1):
            for j in range(_S):
                p = j % 2
                if j < _S - 1:
                    _steady((q, j, p), (q, j + 1))
                elif q == 0:
                    _steady((q, j, p), (1, 0), load=s + 1)
                else:
                    _steady((q, j, p), (0, 0), load=s + 2)
        return carry

    lax.fori_loop(0, (_NSG - 1) // 2, _sg_pair, 0)

    for j in range(_S - 1):
        _steady((0, j, j % 2), (0, j + 1))
    _wait_gathers(0, _S - 1, (_S - 1) % 2)
    _scatters(0, _S - 1, (_S - 1) % 2)
    plsc.subcore_barrier()

    pltpu.sync_copy(acc.at[pl.ds(tbase, _RPT)],
                    out_hbm.at[pl.ds(cid * _NPAD + tbase, _RPT)])


@functools.cache
def _sc_agg():
    return pl.kernel(
        _sc_body,
        out_type=jax.ShapeDtypeStruct((_NC * _NPAD, _H), jnp.float32),
        mesh=plsc.VectorSubcoreMesh(
            core_axis_name="c", subcore_axis_name="s",
            num_cores=_NC, num_subcores=_NS),
        scratch_types=[
            pltpu.VMEM((2, _S, _NB, 2, _G), jnp.int32),
            pltpu.VMEM((2 * _NB * _G, _H), jnp.float32),
            pltpu.VMEM_SHARED((_NPAD, _H), jnp.float32),
            pltpu.SemaphoreType.DMA((2 * _NB,)),
        ],
        compiler_params=pltpu.CompilerParams(use_tc_tiling_on_sc=False),
    )


def _tc_body(x_ref, a0_ref, a1_ref, lo_ref, hi_ref, wl_ref, wh_ref, ws_ref,
             b_ref, o_ref):
    deg = jnp.maximum(hi_ref[...] - lo_ref[...], 1).astype(jnp.float32)
    inv = 1.0 / deg[:, None]
    o_ref[...] = (
        jnp.dot(a0_ref[...] * inv, wl_ref[...], preferred_element_type=jnp.float32)
        + jnp.dot(a1_ref[...] * inv, wh_ref[...], preferred_element_type=jnp.float32)
        + jnp.dot(x_ref[...], ws_ref[...], preferred_element_type=jnp.float32)
        + b_ref[...]
    )


def _tc_combine(x_pad, accs, lo, hi, wl, wh, ws, b2):
    grid = (_NPAD // _R,)
    return pl.pallas_call(
        _tc_body,
        grid=grid,
        in_specs=[
            pl.BlockSpec((_R, _D), lambda i: (i, 0)),
            pl.BlockSpec((_R, _H), lambda i: (i, 0)),
            pl.BlockSpec((_R, _H), lambda i: (i + _NPAD // _R, 0)),
            pl.BlockSpec((_R,), lambda i: (i,)),
            pl.BlockSpec((_R,), lambda i: (i,)),
            pl.BlockSpec((_H, _OUT), lambda i: (0, 0)),
            pl.BlockSpec((_H, _OUT), lambda i: (0, 0)),
            pl.BlockSpec((_D, _OUT), lambda i: (0, 0)),
            pl.BlockSpec((1, _OUT), lambda i: (0, 0)),
        ],
        out_specs=pl.BlockSpec((_R, _OUT), lambda i: (i, 0)),
        out_shape=jax.ShapeDtypeStruct((_NPAD, _OUT), jnp.float32),
    )(x_pad, accs, accs, lo, hi, wl, wh, ws, b2)


def kernel(x_feat, csr_row_ptr, csr_col_ind, unused, sample_count, W, lin_b, bias):
    x_pad = jnp.zeros((_NPAD, _D), jnp.float32).at[:_N].set(x_feat)
    xh = jnp.stack([x_pad[:, :_H], x_pad[:, _H:]])
    seg = jnp.searchsorted(csr_row_ptr[1:], jnp.arange(_E, dtype=jnp.int32),
                           side="right").astype(jnp.int32)
    idx = jnp.stack([csr_col_ind.reshape(_NS, _NSG, _S, _NB, _G),
                     seg.reshape(_NS, _NSG, _S, _NB, _G)], axis=4)

    accs = _sc_agg()(xh, idx)

    lo = jnp.zeros((_NPAD,), jnp.int32).at[:_N].set(csr_row_ptr[:-1])
    hi = jnp.zeros((_NPAD,), jnp.int32).at[:_N].set(csr_row_ptr[1:])
    wl = W[:, :_H].T
    wh = W[:, _H:_D].T
    ws = W[:, _D:].T
    b2 = (lin_b + bias).reshape(1, _OUT)
    y = _tc_combine(x_pad, accs, lo, hi, wl, wh, ws, b2)
    return y[:_N]


# speedup vs baseline: 73.1814x; 73.1814x over previous
import functools

import jax
import jax.numpy as jnp
from jax import lax
from jax.experimental import pallas as pl
from jax.experimental.pallas import tpu as pltpu
from jax.experimental.pallas import tpu_sc as plsc

_N = 10000
_E = 320000
_D = 128
_H = _D // 2
_OUT = 128

_NC = 2
_NS = 16

_R = 1024
_NPAD = ((_N + _R - 1) // _R) * _R
_G = 40
_NB = 5
_GRP = _NB * _G
_S = 4
_EPT = _E // _NS
_NKG = _EPT // _GRP
_NSG = _NKG // _S
_RPT = _NPAD // _NS


def _sc_body(xh_hbm, idx_hbm, out_hbm, idxb, rows, acc, gsem):
    cid = lax.axis_index("c")
    sid = lax.axis_index("s")
    xh = xh_hbm.at[cid]

    z16 = jnp.zeros((16,), jnp.float32)

    def _zrow(r, carry):
        def _zcol(c, inner):
            rows[r, pl.ds(c * 16, 16)] = z16
            return inner
        return lax.fori_loop(0, _H // 16, _zcol, carry)

    lax.fori_loop(0, _G, _zrow, 0)

    tbase = sid * _RPT

    def _zacc(k, carry):
        pltpu.sync_copy(rows.at[pl.ds(0, _G)], acc.at[pl.ds(tbase + k * _G, _G)])
        return carry

    lax.fori_loop(0, _RPT // _G, _zacc, 0)
    plsc.subcore_barrier()

    def _slot(p, b):
        return rows.at[pl.ds((p * _NB + b) * _G, _G)]

    def _ld_sg(s, q):
        pltpu.sync_copy(idx_hbm.at[sid, s], idxb.at[q])

    def _gathers(q, j, p):
        for b in range(_NB):
            pltpu.async_copy(xh.at[idxb.at[q, j, b, 0]], _slot(p, b),
                             gsem.at[p * _NB + b])

    def _wait_gathers(q, j, p):
        for b in range(_NB):
            pltpu.make_async_copy(xh.at[idxb.at[q, j, b, 0]], _slot(p, b),
                                  gsem.at[p * _NB + b]).wait()

    def _scatters(q, j, p):
        for b in range(_NB):
            pltpu.sync_copy(_slot(p, b), acc.at[idxb.at[q, j, b, 1]], add=True)

    def _steady(cur, nxt, load=None):
        (q, j, p), (q2, j2) = cur, nxt
        if load is not None:
            _ld_sg(load, q2)
        _gathers(q2, j2, 1 - p)
        _wait_gathers(q, j, p)
        _scatters(q, j, p)

    _ld_sg(0, 0)
    _gathers(0, 0, 0)

    def _sg_pair(k, carry):
        s = 2 * k
        for q in (0, 1):
            for j in range(_S):
                p = j % 2
                if j < _S - 1:
                    _steady((q, j, p), (q, j + 1))
                elif q == 0:
                    _steady((q, j, p), (1, 0), load=s + 1)
                else:
                    _steady((q, j, p), (0, 0), load=s + 2)
        return carry

    lax.fori_loop(0, (_NSG - 1) // 2, _sg_pair, 0)

    for j in range(_S - 1):
        _steady((0, j, j % 2), (0, j + 1))
    _wait_gathers(0, _S - 1, (_S - 1) % 2)
    _scatters(0, _S - 1, (_S - 1) % 2)
    plsc.subcore_barrier()

    pltpu.sync_copy(acc.at[pl.ds(tbase, _RPT)],
                    out_hbm.at[pl.ds(cid * _NPAD + tbase, _RPT)])


@functools.cache
def _sc_agg():
    return pl.kernel(
        _sc_body,
        out_type=jax.ShapeDtypeStruct((_NC * _NPAD, _H), jnp.float32),
        mesh=plsc.VectorSubcoreMesh(
            core_axis_name="c", subcore_axis_name="s",
            num_cores=_NC, num_subcores=_NS),
        scratch_types=[
            pltpu.VMEM((2, _S, _NB, 2, _G), jnp.int32),
            pltpu.VMEM((2 * _NB * _G, _H), jnp.float32),
            pltpu.VMEM_SHARED((_NPAD, _H), jnp.float32),
            pltpu.SemaphoreType.DMA((2 * _NB,)),
        ],
        compiler_params=pltpu.CompilerParams(use_tc_tiling_on_sc=False),
    )


def _tc_body(x_ref, a0_ref, a1_ref, lo_ref, hi_ref, wl_ref, wh_ref, ws_ref,
             b_ref, o_ref):
    deg = jnp.maximum(hi_ref[...] - lo_ref[...], 1).astype(jnp.float32)
    inv = 1.0 / deg[:, None]
    o_ref[...] = (
        jnp.dot(a0_ref[...] * inv, wl_ref[...], preferred_element_type=jnp.float32)
        + jnp.dot(a1_ref[...] * inv, wh_ref[...], preferred_element_type=jnp.float32)
        + jnp.dot(x_ref[...], ws_ref[...], preferred_element_type=jnp.float32)
        + b_ref[...]
    )


def _tc_combine(x_pad, accs, lo, hi, wl, wh, ws, b2):
    grid = (_NPAD // _R,)
    return pl.pallas_call(
        _tc_body,
        grid=grid,
        in_specs=[
            pl.BlockSpec((_R, _D), lambda i: (i, 0)),
            pl.BlockSpec((_R, _H), lambda i: (i, 0)),
            pl.BlockSpec((_R, _H), lambda i: (i + _NPAD // _R, 0)),
            pl.BlockSpec((_R,), lambda i: (i,)),
            pl.BlockSpec((_R,), lambda i: (i,)),
            pl.BlockSpec((_H, _OUT), lambda i: (0, 0)),
            pl.BlockSpec((_H, _OUT), lambda i: (0, 0)),
            pl.BlockSpec((_D, _OUT), lambda i: (0, 0)),
            pl.BlockSpec((1, _OUT), lambda i: (0, 0)),
        ],
        out_specs=pl.BlockSpec((_R, _OUT), lambda i: (i, 0)),
        out_shape=jax.ShapeDtypeStruct((_NPAD, _OUT), jnp.float32),
    )(x_pad, accs, accs, lo, hi, wl, wh, ws, b2)


def kernel(x_feat, csr_row_ptr, csr_col_ind, unused, sample_count, W, lin_b, bias):
    x_pad = jnp.zeros((_NPAD, _D), jnp.float32).at[:_N].set(x_feat)
    xh = jnp.stack([x_pad[:, :_H], x_pad[:, _H:]])
    marks = jnp.zeros((_E,), jnp.int32).at[csr_row_ptr[1:-1]].add(
        1, mode="drop", indices_are_sorted=True)
    seg = jnp.cumsum(marks, dtype=jnp.int32)
    idx = jnp.stack([csr_col_ind.reshape(_NS, _NSG, _S, _NB, _G),
                     seg.reshape(_NS, _NSG, _S, _NB, _G)], axis=4)

    accs = _sc_agg()(xh, idx)

    lo = jnp.zeros((_NPAD,), jnp.int32).at[:_N].set(csr_row_ptr[:-1])
    hi = jnp.zeros((_NPAD,), jnp.int32).at[:_N].set(csr_row_ptr[1:])
    wl = W[:, :_H].T
    wh = W[:, _H:_D].T
    ws = W[:, _D:].T
    b2 = (lin_b + bias).reshape(1, _OUT)
    y = _tc_combine(x_pad, accs, lo, hi, wl, wh, ws, b2)
    return y[:_N]


# speedup vs baseline: 86.2481x; 1.1786x over previous
import functools

import jax
import jax.numpy as jnp
from jax import lax
from jax.experimental import pallas as pl
from jax.experimental.pallas import tpu as pltpu
from jax.experimental.pallas import tpu_sc as plsc

_N = 10000
_E = 320000
_D = 128
_H = _D // 2
_OUT = 128

_NC = 2
_NS = 16

_R = 1024
_NPAD = ((_N + _R - 1) // _R) * _R
_G = 40
_NB = 5
_GRP = _NB * _G
_S = 4
_EPT = _E // _NS
_NKG = _EPT // _GRP
_NSG = _NKG // _S
_RPT = _NPAD // _NS


def _sc_body(xh_hbm, idx_hbm, out_hbm, idxb, rows, acc, gsem):
    cid = lax.axis_index("c")
    sid = lax.axis_index("s")
    xh = xh_hbm.at[cid]

    z16 = jnp.zeros((16,), jnp.float32)

    def _zrow(r, carry):
        def _zcol(c, inner):
            rows[r, pl.ds(c * 16, 16)] = z16
            return inner
        return lax.fori_loop(0, _H // 16, _zcol, carry)

    lax.fori_loop(0, _G, _zrow, 0)

    tbase = sid * _RPT

    def _zacc(k, carry):
        pltpu.sync_copy(rows.at[pl.ds(0, _G)], acc.at[pl.ds(tbase + k * _G, _G)])
        return carry

    lax.fori_loop(0, _RPT // _G, _zacc, 0)
    plsc.subcore_barrier()

    def _slot(p, b):
        return rows.at[pl.ds((p * _NB + b) * _G, _G)]

    def _ld_sg(s, q):
        pltpu.sync_copy(idx_hbm.at[sid, s], idxb.at[q])

    def _gathers(q, j, p):
        for b in range(_NB):
            pltpu.async_copy(xh.at[idxb.at[q, j, b, 0]], _slot(p, b),
                             gsem.at[p * _NB + b])

    def _wait_gathers(q, j, p):
        for b in range(_NB):
            pltpu.make_async_copy(xh.at[idxb.at[q, j, b, 0]], _slot(p, b),
                                  gsem.at[p * _NB + b]).wait()

    def _scatters(q, j, p):
        for b in range(_NB):
            pltpu.sync_copy(_slot(p, b), acc.at[idxb.at[q, j, b, 1]], add=True)

    def _steady(cur, nxt, load=None):
        (q, j, p), (q2, j2) = cur, nxt
        if load is not None:
            _ld_sg(load, q2)
        _gathers(q2, j2, 1 - p)
        _wait_gathers(q, j, p)
        _scatters(q, j, p)

    _ld_sg(0, 0)
    _gathers(0, 0, 0)

    def _sg_pair(k, carry):
        s = 2 * k
        for q in (0, 1):
            for j in range(_S):
                p = j % 2
                if j < _S - 1:
                    _steady((q, j, p), (q, j + 1))
                elif q == 0:
                    _steady((q, j, p), (1, 0), load=s + 1)
                else:
                    _steady((q, j, p), (0, 0), load=s + 2)
        return carry

    lax.fori_loop(0, (_NSG - 1) // 2, _sg_pair, 0)

    for j in range(_S - 1):
        _steady((0, j, j % 2), (0, j + 1))
    _wait_gathers(0, _S - 1, (_S - 1) % 2)
    _scatters(0, _S - 1, (_S - 1) % 2)
    plsc.subcore_barrier()

    pltpu.sync_copy(acc.at[pl.ds(tbase, _RPT)],
                    out_hbm.at[pl.ds(cid * _NPAD + tbase, _RPT)])


_PW = 384
_PP = 12288
_MP = _E + 256
_MT = _MP // _NS


def _mk_body(ptr_hbm, out_hbm, ptrv, ones, zbuf, marks):
    cid = lax.axis_index("c")
    sid = lax.axis_index("s")
    w = sid * _NC + cid

    z16 = jnp.zeros((16,), jnp.int32)
    lane = jax.lax.iota(jnp.int32, 16)

    def _zb(r, carry):
        zbuf[pl.ds(r * 16, 16)] = z16
        return carry

    lax.fori_loop(0, _MT // 16, _zb, 0)
    ones[pl.ds(0, 16)] = jnp.ones((16,), jnp.int32)

    tbase = sid * _MT
    pltpu.sync_copy(zbuf, marks.at[pl.ds(tbase, _MT)])
    plsc.subcore_barrier()

    pltpu.sync_copy(ptr_hbm.at[pl.ds(w * _PW, _PW)], ptrv)
    for k in range(_PW // 16):
        v = ptrv[pl.ds(k * 16, 16)]
        g = lane + (w * _PW + k * 16)
        v = jnp.where(g == 0, _E, v)
        pltpu.sync_copy(ones.at[pl.ds(0, 16)], marks.at[v], add=True)
    plsc.subcore_barrier()

    pltpu.sync_copy(marks.at[pl.ds(tbase, _MT)],
                    out_hbm.at[pl.ds(cid * _MP + tbase, _MT)])


@functools.cache
def _sc_marks():
    return pl.kernel(
        _mk_body,
        out_type=jax.ShapeDtypeStruct((_NC * _MP,), jnp.int32),
        mesh=plsc.VectorSubcoreMesh(
            core_axis_name="c", subcore_axis_name="s",
            num_cores=_NC, num_subcores=_NS),
        scratch_types=[
            pltpu.VMEM((_PW,), jnp.int32),
            pltpu.VMEM((16,), jnp.int32),
            pltpu.VMEM((_MT,), jnp.int32),
            pltpu.VMEM_SHARED((_MP,), jnp.int32),
        ],
        compiler_params=pltpu.CompilerParams(use_tc_tiling_on_sc=False),
    )


@functools.cache
def _sc_agg():
    return pl.kernel(
        _sc_body,
        out_type=jax.ShapeDtypeStruct((_NC * _NPAD, _H), jnp.float32),
        mesh=plsc.VectorSubcoreMesh(
            core_axis_name="c", subcore_axis_name="s",
            num_cores=_NC, num_subcores=_NS),
        scratch_types=[
            pltpu.VMEM((2, _S, _NB, 2, _G), jnp.int32),
            pltpu.VMEM((2 * _NB * _G, _H), jnp.float32),
            pltpu.VMEM_SHARED((_NPAD, _H), jnp.float32),
            pltpu.SemaphoreType.DMA((2 * _NB,)),
        ],
        compiler_params=pltpu.CompilerParams(use_tc_tiling_on_sc=False),
    )


def _tc_body(x_ref, a0_ref, a1_ref, lo_ref, hi_ref, wl_ref, wh_ref, ws_ref,
             b_ref, o_ref):
    deg = jnp.maximum(hi_ref[...] - lo_ref[...], 1).astype(jnp.float32)
    inv = 1.0 / deg[:, None]
    o_ref[...] = (
        jnp.dot(a0_ref[...] * inv, wl_ref[...], preferred_element_type=jnp.float32)
        + jnp.dot(a1_ref[...] * inv, wh_ref[...], preferred_element_type=jnp.float32)
        + jnp.dot(x_ref[...], ws_ref[...], preferred_element_type=jnp.float32)
        + b_ref[...]
    )


def _tc_combine(x_pad, accs, lo, hi, wl, wh, ws, b2):
    grid = (_NPAD // _R,)
    return pl.pallas_call(
        _tc_body,
        grid=grid,
        in_specs=[
            pl.BlockSpec((_R, _D), lambda i: (i, 0)),
            pl.BlockSpec((_R, _H), lambda i: (i, 0)),
            pl.BlockSpec((_R, _H), lambda i: (i + _NPAD // _R, 0)),
            pl.BlockSpec((_R,), lambda i: (i,)),
            pl.BlockSpec((_R,), lambda i: (i,)),
            pl.BlockSpec((_H, _OUT), lambda i: (0, 0)),
            pl.BlockSpec((_H, _OUT), lambda i: (0, 0)),
            pl.BlockSpec((_D, _OUT), lambda i: (0, 0)),
            pl.BlockSpec((1, _OUT), lambda i: (0, 0)),
        ],
        out_specs=pl.BlockSpec((_R, _OUT), lambda i: (i, 0)),
        out_shape=jax.ShapeDtypeStruct((_NPAD, _OUT), jnp.float32),
    )(x_pad, accs, accs, lo, hi, wl, wh, ws, b2)


def kernel(x_feat, csr_row_ptr, csr_col_ind, unused, sample_count, W, lin_b, bias):
    x_pad = jnp.zeros((_NPAD, _D), jnp.float32).at[:_N].set(x_feat)
    xh = jnp.stack([x_pad[:, :_H], x_pad[:, _H:]])
    ptr_pad = jnp.full((_PP,), _E, jnp.int32).at[:_N + 1].set(csr_row_ptr)
    m2 = _sc_marks()(ptr_pad)
    marks = m2[:_E] + m2[_MP:_MP + _E]
    seg = jnp.cumsum(marks, dtype=jnp.int32)
    idx = jnp.stack([csr_col_ind.reshape(_NS, _NSG, _S, _NB, _G),
                     seg.reshape(_NS, _NSG, _S, _NB, _G)], axis=4)

    accs = _sc_agg()(xh, idx)

    lo = jnp.zeros((_NPAD,), jnp.int32).at[:_N].set(csr_row_ptr[:-1])
    hi = jnp.zeros((_NPAD,), jnp.int32).at[:_N].set(csr_row_ptr[1:])
    wl = W[:, :_H].T
    wh = W[:, _H:_D].T
    ws = W[:, _D:].T
    b2 = (lin_b + bias).reshape(1, _OUT)
    y = _tc_combine(x_pad, accs, lo, hi, wl, wh, ws, b2)
    return y[:_N]


# speedup vs baseline: 110.9927x; 1.2869x over previous
import functools

import jax
import jax.numpy as jnp
from jax import lax
from jax.experimental import pallas as pl
from jax.experimental.pallas import tpu as pltpu
from jax.experimental.pallas import tpu_sc as plsc

_N = 10000
_E = 320000
_D = 128
_H = _D // 2
_OUT = 128

_NC = 2
_NS = 16

_R = 1024
_NPAD = ((_N + _R - 1) // _R) * _R
_G = 40
_NB = 5
_GRP = _NB * _G
_S = 4
_EPT = _E // _NS
_NKG = _EPT // _GRP
_NSG = _NKG // _S
_RPT = _NPAD // _NS


def _sc_body(x2_hbm, col_hbm, seg_hbm, out_hbm, colb, segb, rows, acc, gsem):
    cid = lax.axis_index("c")
    sid = lax.axis_index("s")

    z16 = jnp.zeros((16,), jnp.float32)

    def _zrow(r, carry):
        def _zcol(c, inner):
            rows[r, pl.ds(c * 16, 16)] = z16
            return inner
        return lax.fori_loop(0, _H // 16, _zcol, carry)

    lax.fori_loop(0, _G, _zrow, 0)

    tbase = sid * _RPT

    def _zacc(k, carry):
        pltpu.sync_copy(rows.at[pl.ds(0, _G)], acc.at[pl.ds(tbase + k * _G, _G)])
        return carry

    lax.fori_loop(0, _RPT // _G, _zacc, 0)
    plsc.subcore_barrier()

    def _slot(p, b):
        return rows.at[pl.ds((p * _NB + b) * _G, _G)]

    def _ld_sg(s, q):
        pltpu.sync_copy(col_hbm.at[sid, s], colb.at[q])
        pltpu.sync_copy(seg_hbm.at[sid, s], segb.at[q])

        def _tr(k, carry):
            v = colb[q, pl.ds(k * 16, 16)]
            colb[q, pl.ds(k * 16, 16)] = v + v + cid
            return carry

        lax.fori_loop(0, (_S * _NB * _G) // 16, _tr, 0)

    def _cref(q, j, b):
        return colb.at[q, pl.ds((j * _NB + b) * _G, _G)]

    def _gathers(q, j, p):
        for b in range(_NB):
            pltpu.async_copy(x2_hbm.at[_cref(q, j, b)], _slot(p, b),
                             gsem.at[p * _NB + b])

    def _wait_gathers(q, j, p):
        for b in range(_NB):
            pltpu.make_async_copy(x2_hbm.at[_cref(q, j, b)], _slot(p, b),
                                  gsem.at[p * _NB + b]).wait()

    def _scatters(q, j, p):
        for b in range(_NB):
            pltpu.sync_copy(_slot(p, b), acc.at[segb.at[q, j, b]], add=True)

    def _steady(cur, nxt, load=None):
        (q, j, p), (q2, j2) = cur, nxt
        if load is not None:
            _ld_sg(load, q2)
        _gathers(q2, j2, 1 - p)
        _wait_gathers(q, j, p)
        _scatters(q, j, p)

    _ld_sg(0, 0)
    _gathers(0, 0, 0)

    def _sg_pair(k, carry):
        s = 2 * k
        for q in (0, 1):
            for j in range(_S):
                p = j % 2
                if j < _S - 1:
                    _steady((q, j, p), (q, j + 1))
                elif q == 0:
                    _steady((q, j, p), (1, 0), load=s + 1)
                else:
                    _steady((q, j, p), (0, 0), load=s + 2)
        return carry

    lax.fori_loop(0, (_NSG - 1) // 2, _sg_pair, 0)

    for j in range(_S - 1):
        _steady((0, j, j % 2), (0, j + 1))
    _wait_gathers(0, _S - 1, (_S - 1) % 2)
    _scatters(0, _S - 1, (_S - 1) % 2)
    plsc.subcore_barrier()

    pltpu.sync_copy(acc.at[pl.ds(tbase, _RPT)],
                    out_hbm.at[pl.ds(cid * _NPAD + tbase, _RPT)])


_PW = 384
_PP = 12288
_MP = _E + 256
_MT = _MP // _NS


def _mk_body(ptr_hbm, out_hbm, ptrv, ones, zbuf, marks):
    cid = lax.axis_index("c")
    sid = lax.axis_index("s")
    w = sid * _NC + cid

    z16 = jnp.zeros((16,), jnp.int32)
    lane = jax.lax.iota(jnp.int32, 16)

    def _zb(r, carry):
        zbuf[pl.ds(r * 16, 16)] = z16
        return carry

    lax.fori_loop(0, _MT // 16, _zb, 0)
    ones[pl.ds(0, 16)] = jnp.ones((16,), jnp.int32)

    tbase = sid * _MT
    pltpu.sync_copy(zbuf, marks.at[pl.ds(tbase, _MT)])
    plsc.subcore_barrier()

    pltpu.sync_copy(ptr_hbm.at[pl.ds(w * _PW, _PW)], ptrv)
    for k in range(_PW // 16):
        v = ptrv[pl.ds(k * 16, 16)]
        g = lane + (w * _PW + k * 16)
        v = jnp.where(g == 0, _E, v)
        pltpu.sync_copy(ones.at[pl.ds(0, 16)], marks.at[v], add=True)
    plsc.subcore_barrier()

    pltpu.sync_copy(marks.at[pl.ds(tbase, _MT)],
                    out_hbm.at[pl.ds(cid * _MP + tbase, _MT)])


@functools.cache
def _sc_marks():
    return pl.kernel(
        _mk_body,
        out_type=jax.ShapeDtypeStruct((_NC * _MP,), jnp.int32),
        mesh=plsc.VectorSubcoreMesh(
            core_axis_name="c", subcore_axis_name="s",
            num_cores=_NC, num_subcores=_NS),
        scratch_types=[
            pltpu.VMEM((_PW,), jnp.int32),
            pltpu.VMEM((16,), jnp.int32),
            pltpu.VMEM((_MT,), jnp.int32),
            pltpu.VMEM_SHARED((_MP,), jnp.int32),
        ],
        compiler_params=pltpu.CompilerParams(use_tc_tiling_on_sc=False),
    )


@functools.cache
def _sc_agg():
    return pl.kernel(
        _sc_body,
        out_type=jax.ShapeDtypeStruct((_NC * _NPAD, _H), jnp.float32),
        mesh=plsc.VectorSubcoreMesh(
            core_axis_name="c", subcore_axis_name="s",
            num_cores=_NC, num_subcores=_NS),
        scratch_types=[
            pltpu.VMEM((2, _S * _NB * _G), jnp.int32),
            pltpu.VMEM((2, _S, _NB, _G), jnp.int32),
            pltpu.VMEM((2 * _NB * _G, _H), jnp.float32),
            pltpu.VMEM_SHARED((_NPAD, _H), jnp.float32),
            pltpu.SemaphoreType.DMA((2 * _NB,)),
        ],
        compiler_params=pltpu.CompilerParams(use_tc_tiling_on_sc=False),
    )


def _tc_body(x_ref, a0_ref, a1_ref, lo_ref, hi_ref, wl_ref, wh_ref, ws_ref,
             b_ref, o_ref):
    deg = jnp.maximum(hi_ref[...] - lo_ref[...], 1).astype(jnp.float32)
    inv = 1.0 / deg[:, None]
    o_ref[...] = (
        jnp.dot(a0_ref[...] * inv, wl_ref[...], preferred_element_type=jnp.float32)
        + jnp.dot(a1_ref[...] * inv, wh_ref[...], preferred_element_type=jnp.float32)
        + jnp.dot(x_ref[...], ws_ref[...], preferred_element_type=jnp.float32)
        + b_ref[...]
    )


def _tc_combine(x_pad, accs, lo, hi, wl, wh, ws, b2):
    grid = (_NPAD // _R,)
    return pl.pallas_call(
        _tc_body,
        grid=grid,
        in_specs=[
            pl.BlockSpec((_R, _D), lambda i: (i, 0)),
            pl.BlockSpec((_R, _H), lambda i: (i, 0)),
            pl.BlockSpec((_R, _H), lambda i: (i + _NPAD // _R, 0)),
            pl.BlockSpec((_R,), lambda i: (i,)),
            pl.BlockSpec((_R,), lambda i: (i,)),
            pl.BlockSpec((_H, _OUT), lambda i: (0, 0)),
            pl.BlockSpec((_H, _OUT), lambda i: (0, 0)),
            pl.BlockSpec((_D, _OUT), lambda i: (0, 0)),
            pl.BlockSpec((1, _OUT), lambda i: (0, 0)),
        ],
        out_specs=pl.BlockSpec((_R, _OUT), lambda i: (i, 0)),
        out_shape=jax.ShapeDtypeStruct((_NPAD, _OUT), jnp.float32),
    )(x_pad, accs, accs, lo, hi, wl, wh, ws, b2)


def kernel(x_feat, csr_row_ptr, csr_col_ind, unused, sample_count, W, lin_b, bias):
    x_pad = jnp.zeros((_NPAD, _D), jnp.float32).at[:_N].set(x_feat)
    x2 = x_feat.reshape(2 * _N, _H)
    ptr_pad = jnp.full((_PP,), _E, jnp.int32).at[:_N + 1].set(csr_row_ptr)
    m2 = _sc_marks()(ptr_pad)
    marks = m2[:_E] + m2[_MP:_MP + _E]
    seg = jnp.cumsum(marks, dtype=jnp.int32)
    col6 = csr_col_ind.reshape(_NS, _NSG, _S * _NB * _G)
    seg6 = seg.reshape(_NS, _NSG, _S, _NB, _G)

    accs = _sc_agg()(x2, col6, seg6)

    lo = jnp.zeros((_NPAD,), jnp.int32).at[:_N].set(csr_row_ptr[:-1])
    hi = jnp.zeros((_NPAD,), jnp.int32).at[:_N].set(csr_row_ptr[1:])
    wl = W[:, :_H].T
    wh = W[:, _H:_D].T
    ws = W[:, _D:].T
    b2 = (lin_b + bias).reshape(1, _OUT)
    y = _tc_combine(x_pad, accs, lo, hi, wl, wh, ws, b2)
    return y[:_N]


# speedup vs baseline: 117.0008x; 1.0541x over previous
import functools

import jax
import jax.numpy as jnp
from jax import lax
from jax.experimental import pallas as pl
from jax.experimental.pallas import tpu as pltpu
from jax.experimental.pallas import tpu_sc as plsc

_N = 10000
_E = 320000
_D = 128
_H = _D // 2
_OUT = 128

_NC = 2
_NS = 16

_R = 1024
_NPAD = ((_N + _R - 1) // _R) * _R
_G = 40
_NB = 5
_GRP = _NB * _G
_S = 4
_EPT = _E // _NS
_NKG = _EPT // _GRP
_NSG = _NKG // _S
_RPT = _NPAD // _NS


def _sc_body(x2_hbm, col_hbm, seg_hbm, out_hbm, colb, segb, rows, acc, gsem):
    cid = lax.axis_index("c")
    sid = lax.axis_index("s")

    z16 = jnp.zeros((16,), jnp.float32)

    def _zrow(r, carry):
        def _zcol(c, inner):
            rows[r, pl.ds(c * 16, 16)] = z16
            return inner
        return lax.fori_loop(0, _H // 16, _zcol, carry)

    lax.fori_loop(0, _G, _zrow, 0)

    tbase = sid * _RPT

    def _zacc(k, carry):
        pltpu.sync_copy(rows.at[pl.ds(0, _G)], acc.at[pl.ds(tbase + k * _G, _G)])
        return carry

    lax.fori_loop(0, _RPT // _G, _zacc, 0)
    plsc.subcore_barrier()

    def _slot(p, b):
        return rows.at[pl.ds((p * _NB + b) * _G, _G)]

    def _ld_sg(s, q):
        pltpu.sync_copy(col_hbm.at[sid, s], colb.at[q])
        pltpu.sync_copy(seg_hbm.at[sid, s], segb.at[q])

        def _tr(k, carry):
            v = colb[q, pl.ds(k * 16, 16)]
            colb[q, pl.ds(k * 16, 16)] = v + v + cid
            return carry

        lax.fori_loop(0, (_S * _NB * _G) // 16, _tr, 0)

    def _cref(q, j, b):
        return colb.at[q, pl.ds((j * _NB + b) * _G, _G)]

    def _gathers(q, j, p):
        for b in range(_NB):
            pltpu.async_copy(x2_hbm.at[_cref(q, j, b)], _slot(p, b),
                             gsem.at[p * _NB + b])

    def _wait_gathers(q, j, p):
        for b in range(_NB):
            pltpu.make_async_copy(x2_hbm.at[_cref(q, j, b)], _slot(p, b),
                                  gsem.at[p * _NB + b]).wait()

    def _scatters(q, j, p):
        for b in range(_NB):
            pltpu.sync_copy(_slot(p, b), acc.at[segb.at[q, j, b]], add=True)

    def _steady(cur, nxt, load=None):
        (q, j, p), (q2, j2) = cur, nxt
        if load is not None:
            _ld_sg(load, q2)
        _gathers(q2, j2, 1 - p)
        _wait_gathers(q, j, p)
        _scatters(q, j, p)

    _ld_sg(0, 0)
    _gathers(0, 0, 0)

    def _sg_pair(k, carry):
        s = 2 * k
        for q in (0, 1):
            for j in range(_S):
                p = j % 2
                if j < _S - 1:
                    _steady((q, j, p), (q, j + 1))
                elif q == 0:
                    _steady((q, j, p), (1, 0), load=s + 1)
                else:
                    _steady((q, j, p), (0, 0), load=s + 2)
        return carry

    lax.fori_loop(0, (_NSG - 1) // 2, _sg_pair, 0)

    for j in range(_S - 1):
        _steady((0, j, j % 2), (0, j + 1))
    _wait_gathers(0, _S - 1, (_S - 1) % 2)
    _scatters(0, _S - 1, (_S - 1) % 2)
    plsc.subcore_barrier()

    pltpu.sync_copy(acc.at[pl.ds(tbase, _RPT)],
                    out_hbm.at[pl.ds(tbase, _RPT), pl.ds(cid * _H, _H)])


_PW = 384
_PP = 12288
_MP = _E + 256
_MT = _MP // _NS


def _mk_body(ptr_hbm, out_hbm, ptrv, ones, zbuf, marks):
    cid = lax.axis_index("c")
    sid = lax.axis_index("s")
    w = sid * _NC + cid

    z16 = jnp.zeros((16,), jnp.int32)
    lane = jax.lax.iota(jnp.int32, 16)

    def _zb(r, carry):
        zbuf[pl.ds(r * 16, 16)] = z16
        return carry

    lax.fori_loop(0, _MT // 16, _zb, 0)
    ones[pl.ds(0, 16)] = jnp.ones((16,), jnp.int32)

    tbase = sid * _MT
    pltpu.sync_copy(zbuf, marks.at[pl.ds(tbase, _MT)])
    plsc.subcore_barrier()

    pltpu.sync_copy(ptr_hbm.at[pl.ds(w * _PW, _PW)], ptrv)
    for k in range(_PW // 16):
        v = ptrv[pl.ds(k * 16, 16)]
        g = lane + (w * _PW + k * 16)
        v = jnp.where(g == 0, _E, v)
        pltpu.sync_copy(ones.at[pl.ds(0, 16)], marks.at[v], add=True)
    plsc.subcore_barrier()

    pltpu.sync_copy(marks.at[pl.ds(tbase, _MT)],
                    out_hbm.at[pl.ds(cid * _MP + tbase, _MT)])


@functools.cache
def _sc_marks():
    return pl.kernel(
        _mk_body,
        out_type=jax.ShapeDtypeStruct((_NC * _MP,), jnp.int32),
        mesh=plsc.VectorSubcoreMesh(
            core_axis_name="c", subcore_axis_name="s",
            num_cores=_NC, num_subcores=_NS),
        scratch_types=[
            pltpu.VMEM((_PW,), jnp.int32),
            pltpu.VMEM((16,), jnp.int32),
            pltpu.VMEM((_MT,), jnp.int32),
            pltpu.VMEM_SHARED((_MP,), jnp.int32),
        ],
        compiler_params=pltpu.CompilerParams(use_tc_tiling_on_sc=False),
    )


@functools.cache
def _sc_agg():
    return pl.kernel(
        _sc_body,
        out_type=jax.ShapeDtypeStruct((_NPAD, _D), jnp.float32),
        mesh=plsc.VectorSubcoreMesh(
            core_axis_name="c", subcore_axis_name="s",
            num_cores=_NC, num_subcores=_NS),
        scratch_types=[
            pltpu.VMEM((2, _S * _NB * _G), jnp.int32),
            pltpu.VMEM((2, _S, _NB, _G), jnp.int32),
            pltpu.VMEM((2 * _NB * _G, _H), jnp.float32),
            pltpu.VMEM_SHARED((_NPAD, _H), jnp.float32),
            pltpu.SemaphoreType.DMA((2 * _NB,)),
        ],
        compiler_params=pltpu.CompilerParams(use_tc_tiling_on_sc=False),
    )


def _tc_body(x_ref, a_ref, lo_ref, hi_ref, wa_ref, ws_ref, b_ref, o_ref):
    deg = jnp.maximum(hi_ref[...] - lo_ref[...], 1).astype(jnp.float32)
    agg = a_ref[...] / deg[:, None]
    o_ref[...] = (
        jnp.dot(agg, wa_ref[...], preferred_element_type=jnp.float32)
        + jnp.dot(x_ref[...], ws_ref[...], preferred_element_type=jnp.float32)
        + b_ref[...]
    )


def _tc_combine(x_pad, accs, lo, hi, wa, ws, b2):
    grid = (_NPAD // _R,)
    return pl.pallas_call(
        _tc_body,
        grid=grid,
        in_specs=[
            pl.BlockSpec((_R, _D), lambda i: (i, 0)),
            pl.BlockSpec((_R, _D), lambda i: (i, 0)),
            pl.BlockSpec((_R,), lambda i: (i,)),
            pl.BlockSpec((_R,), lambda i: (i,)),
            pl.BlockSpec((_D, _OUT), lambda i: (0, 0)),
            pl.BlockSpec((_D, _OUT), lambda i: (0, 0)),
            pl.BlockSpec((1, _OUT), lambda i: (0, 0)),
        ],
        out_specs=pl.BlockSpec((_R, _OUT), lambda i: (i, 0)),
        out_shape=jax.ShapeDtypeStruct((_NPAD, _OUT), jnp.float32),
    )(x_pad, accs, lo, hi, wa, ws, b2)


def kernel(x_feat, csr_row_ptr, csr_col_ind, unused, sample_count, W, lin_b, bias):
    x_pad = jnp.zeros((_NPAD, _D), jnp.float32).at[:_N].set(x_feat)
    x2 = x_feat.reshape(2 * _N, _H)
    ptr_pad = jnp.full((_PP,), _E, jnp.int32).at[:_N + 1].set(csr_row_ptr)
    m2 = _sc_marks()(ptr_pad)
    marks = m2[:_E] + m2[_MP:_MP + _E]
    seg = jnp.cumsum(marks, dtype=jnp.int32)
    col6 = csr_col_ind.reshape(_NS, _NSG, _S * _NB * _G)
    seg6 = seg.reshape(_NS, _NSG, _S, _NB, _G)

    accs = _sc_agg()(x2, col6, seg6)

    lo = jnp.zeros((_NPAD,), jnp.int32).at[:_N].set(csr_row_ptr[:-1])
    hi = jnp.zeros((_NPAD,), jnp.int32).at[:_N].set(csr_row_ptr[1:])
    wa = W[:, :_D].T
    ws = W[:, _D:].T
    b2 = (lin_b + bias).reshape(1, _OUT)
    y = _tc_combine(x_pad, accs, lo, hi, wa, ws, b2)
    return y[:_N]
